# baseline (speedup 1.0000x reference)
import functools

import jax
import jax.numpy as jnp
from jax.experimental import pallas as pl

_N = 10000
_D = 128
_ALPHA = 0.2
_BM = 200


def _support_body(x_ref, w_ref, o_ref):
    o_ref[...] = jnp.dot(x_ref[...], w_ref[...],
                         preferred_element_type=jnp.float32)


def _support(x, w):
    return pl.pallas_call(
        _support_body,
        out_shape=jax.ShapeDtypeStruct((_N, _D), jnp.float32),
    )(x, w)


def _agg_chain_body(adj_ref, s_ref, b_ref, w2_ref, o_ref):
    z = jnp.dot(adj_ref[...], s_ref[...],
                preferred_element_type=jnp.float32) + b_ref[...]
    z = jnp.where(z >= 0, z, _ALPHA * z)
    o_ref[...] = jnp.dot(z, w2_ref[...], preferred_element_type=jnp.float32)


def _agg_chain(adj, s, b, w2):
    grid = (_N // _BM,)
    return pl.pallas_call(
        _agg_chain_body,
        grid=grid,
        in_specs=[
            pl.BlockSpec((_BM, _N), lambda i: (i, 0)),
            pl.BlockSpec((_N, _D), lambda i: (0, 0)),
            pl.BlockSpec((1, _D), lambda i: (0, 0)),
            pl.BlockSpec((_D, _D), lambda i: (0, 0)),
        ],
        out_specs=pl.BlockSpec((_BM, _D), lambda i: (i, 0)),
        out_shape=jax.ShapeDtypeStruct((_N, _D), jnp.float32),
    )(adj, s, b.reshape(1, _D), w2)


def _agg_relu_sup_body(adj_ref, s_ref, b_ref, w2_ref, act_ref, sup_ref):
    z = jnp.dot(adj_ref[...], s_ref[...],
                preferred_element_type=jnp.float32) + b_ref[...]
    z = jnp.maximum(z, 0.0)
    act_ref[...] = z
    sup_ref[...] = jnp.dot(z, w2_ref[...], preferred_element_type=jnp.float32)


def _agg_relu_sup(adj, s, b, w2):
    grid = (_N // _BM,)
    return pl.pallas_call(
        _agg_relu_sup_body,
        grid=grid,
        in_specs=[
            pl.BlockSpec((_BM, _N), lambda i: (i, 0)),
            pl.BlockSpec((_N, _D), lambda i: (0, 0)),
            pl.BlockSpec((1, _D), lambda i: (0, 0)),
            pl.BlockSpec((_D, _D), lambda i: (0, 0)),
        ],
        out_specs=[
            pl.BlockSpec((_BM, _D), lambda i: (i, 0)),
            pl.BlockSpec((_BM, _D), lambda i: (i, 0)),
        ],
        out_shape=[
            jax.ShapeDtypeStruct((_N, _D), jnp.float32),
            jax.ShapeDtypeStruct((_N, _D), jnp.float32),
        ],
    )(adj, s, b.reshape(1, _D), w2)


def _agg_mean_body(adj_ref, s_ref, b_ref, x0_ref, x1_ref, o_ref):
    z = jnp.dot(adj_ref[...], s_ref[...],
                preferred_element_type=jnp.float32) + b_ref[...]
    z = jnp.maximum(z, 0.0)
    o_ref[...] = (x0_ref[...] + x1_ref[...] + z) * (1.0 / 3.0)


def _agg_mean(adj, s, b, x0, x1):
    grid = (_N // _BM,)
    return pl.pallas_call(
        _agg_mean_body,
        grid=grid,
        in_specs=[
            pl.BlockSpec((_BM, _N), lambda i: (i, 0)),
            pl.BlockSpec((_N, _D), lambda i: (0, 0)),
            pl.BlockSpec((1, _D), lambda i: (0, 0)),
            pl.BlockSpec((_BM, _D), lambda i: (i, 0)),
            pl.BlockSpec((_BM, _D), lambda i: (i, 0)),
        ],
        out_specs=pl.BlockSpec((_BM, _D), lambda i: (i, 0)),
        out_shape=jax.ShapeDtypeStruct((_N, _D), jnp.float32),
    )(adj, s, b.reshape(1, _D), x0, x1)


def kernel(ufea, vfea, UV_adj, VU_adj, params):
    p = params
    s_u = _support(ufea, p["W_0_0"])
    s_v = _support(vfea, p["W_0_1"])
    s_u = _agg_chain(VU_adj, s_u, p["b_0_0"], p["W_0_2"])
    s_v = _agg_chain(UV_adj, s_v, p["b_0_1"], p["W_0_3"])
    u1, s_u = _agg_relu_sup(UV_adj, s_u, p["b_0_2"], p["W_1_0"])
    v1, s_v = _agg_relu_sup(VU_adj, s_v, p["b_0_3"], p["W_1_1"])
    s_u = _agg_chain(VU_adj, s_u, p["b_1_0"], p["W_1_2"])
    s_v = _agg_chain(UV_adj, s_v, p["b_1_1"], p["W_1_3"])
    out_u = _agg_mean(UV_adj, s_u, p["b_1_2"], ufea, u1)
    out_v = _agg_mean(VU_adj, s_v, p["b_1_3"], vfea, v1)
    return (out_u, out_v)


# speedup vs baseline: 1.0827x; 1.0827x over previous
import functools

import jax
import jax.numpy as jnp
from jax.experimental import pallas as pl

_N = 10000
_D = 128
_ALPHA = 0.2
_BM = 200


def _support_body(x_ref, w_ref, o_ref):
    o_ref[...] = jnp.dot(x_ref[...], w_ref[...],
                         preferred_element_type=jnp.float32).astype(jnp.bfloat16)


def _support(x, w):
    return pl.pallas_call(
        _support_body,
        out_shape=jax.ShapeDtypeStruct((_N, _D), jnp.bfloat16),
    )(x, w)


def _leaky(z):
    return jnp.where(z >= 0, z, _ALPHA * z)


def _agg_conv_body(adj_ref, s_ref, b_ref, w2_ref, o_ref, adj16_ref):
    a16 = adj_ref[...].astype(jnp.bfloat16)
    adj16_ref[...] = a16
    z = jnp.dot(a16, s_ref[...], preferred_element_type=jnp.float32) + b_ref[...]
    z = _leaky(z)
    o_ref[...] = jnp.dot(z, w2_ref[...],
                         preferred_element_type=jnp.float32).astype(jnp.bfloat16)


def _agg_conv(adj, s, b, w2):
    return pl.pallas_call(
        _agg_conv_body,
        grid=(_N // _BM,),
        in_specs=[
            pl.BlockSpec((_BM, _N), lambda i: (i, 0)),
            pl.BlockSpec((_N, _D), lambda i: (0, 0)),
            pl.BlockSpec((1, _D), lambda i: (0, 0)),
            pl.BlockSpec((_D, _D), lambda i: (0, 0)),
        ],
        out_specs=[
            pl.BlockSpec((_BM, _D), lambda i: (i, 0)),
            pl.BlockSpec((_BM, _N), lambda i: (i, 0)),
        ],
        out_shape=[
            jax.ShapeDtypeStruct((_N, _D), jnp.bfloat16),
            jax.ShapeDtypeStruct((_N, _N), jnp.bfloat16),
        ],
    )(adj, s, b.reshape(1, _D), w2)


def _agg_relu_sup_body(adj_ref, s_ref, b_ref, w2_ref, act_ref, sup_ref):
    z = jnp.dot(adj_ref[...], s_ref[...],
                preferred_element_type=jnp.float32) + b_ref[...]
    z = jnp.maximum(z, 0.0)
    act_ref[...] = z
    sup_ref[...] = jnp.dot(z, w2_ref[...],
                           preferred_element_type=jnp.float32).astype(jnp.bfloat16)


def _agg_relu_sup(adj, s, b, w2):
    return pl.pallas_call(
        _agg_relu_sup_body,
        grid=(_N // _BM,),
        in_specs=[
            pl.BlockSpec((_BM, _N), lambda i: (i, 0)),
            pl.BlockSpec((_N, _D), lambda i: (0, 0)),
            pl.BlockSpec((1, _D), lambda i: (0, 0)),
            pl.BlockSpec((_D, _D), lambda i: (0, 0)),
        ],
        out_specs=[
            pl.BlockSpec((_BM, _D), lambda i: (i, 0)),
            pl.BlockSpec((_BM, _D), lambda i: (i, 0)),
        ],
        out_shape=[
            jax.ShapeDtypeStruct((_N, _D), jnp.float32),
            jax.ShapeDtypeStruct((_N, _D), jnp.bfloat16),
        ],
    )(adj, s, b.reshape(1, _D), w2)


def _agg_chain_body(adj_ref, s_ref, b_ref, w2_ref, o_ref):
    z = jnp.dot(adj_ref[...], s_ref[...],
                preferred_element_type=jnp.float32) + b_ref[...]
    z = _leaky(z)
    o_ref[...] = jnp.dot(z, w2_ref[...],
                         preferred_element_type=jnp.float32).astype(jnp.bfloat16)


def _agg_chain(adj, s, b, w2):
    return pl.pallas_call(
        _agg_chain_body,
        grid=(_N // _BM,),
        in_specs=[
            pl.BlockSpec((_BM, _N), lambda i: (i, 0)),
            pl.BlockSpec((_N, _D), lambda i: (0, 0)),
            pl.BlockSpec((1, _D), lambda i: (0, 0)),
            pl.BlockSpec((_D, _D), lambda i: (0, 0)),
        ],
        out_specs=pl.BlockSpec((_BM, _D), lambda i: (i, 0)),
        out_shape=jax.ShapeDtypeStruct((_N, _D), jnp.bfloat16),
    )(adj, s, b.reshape(1, _D), w2)


def _agg_mean_body(adj_ref, s_ref, b_ref, x0_ref, x1_ref, o_ref):
    z = jnp.dot(adj_ref[...], s_ref[...],
                preferred_element_type=jnp.float32) + b_ref[...]
    z = jnp.maximum(z, 0.0)
    o_ref[...] = (x0_ref[...] + x1_ref[...] + z) * (1.0 / 3.0)


def _agg_mean(adj, s, b, x0, x1):
    return pl.pallas_call(
        _agg_mean_body,
        grid=(_N // _BM,),
        in_specs=[
            pl.BlockSpec((_BM, _N), lambda i: (i, 0)),
            pl.BlockSpec((_N, _D), lambda i: (0, 0)),
            pl.BlockSpec((1, _D), lambda i: (0, 0)),
            pl.BlockSpec((_BM, _D), lambda i: (i, 0)),
            pl.BlockSpec((_BM, _D), lambda i: (i, 0)),
        ],
        out_specs=pl.BlockSpec((_BM, _D), lambda i: (i, 0)),
        out_shape=jax.ShapeDtypeStruct((_N, _D), jnp.float32),
    )(adj, s, b.reshape(1, _D), x0, x1)


def kernel(ufea, vfea, UV_adj, VU_adj, params):
    p = params
    s_u = _support(ufea, p["W_0_0"])
    s_v = _support(vfea, p["W_0_1"])
    s_u, VU16 = _agg_conv(VU_adj, s_u, p["b_0_0"], p["W_0_2"])
    s_v, UV16 = _agg_conv(UV_adj, s_v, p["b_0_1"], p["W_0_3"])
    u1, s_u = _agg_relu_sup(UV16, s_u, p["b_0_2"], p["W_1_0"])
    v1, s_v = _agg_relu_sup(VU16, s_v, p["b_0_3"], p["W_1_1"])
    s_u = _agg_chain(VU16, s_u, p["b_1_0"], p["W_1_2"])
    s_v = _agg_chain(UV16, s_v, p["b_1_1"], p["W_1_3"])
    out_u = _agg_mean(UV16, s_u, p["b_1_2"], ufea, u1)
    out_v = _agg_mean(VU16, s_v, p["b_1_3"], vfea, v1)
    return (out_u, out_v)


# speedup vs baseline: 1.2462x; 1.1511x over previous
import functools

import jax
import jax.numpy as jnp
from jax.experimental import pallas as pl

_N = 10000
_D = 128
_ALPHA = 0.2
_BM = 400


def _support_body(x_ref, w_ref, o_ref):
    o_ref[...] = jnp.dot(x_ref[...], w_ref[...],
                         preferred_element_type=jnp.float32).astype(jnp.bfloat16)


def _support(x, w):
    return pl.pallas_call(
        _support_body,
        out_shape=jax.ShapeDtypeStruct((_N, _D), jnp.bfloat16),
    )(x, w)


def _leaky(z):
    return jnp.where(z >= 0, z, _ALPHA * z)


def _agg_conv_body(adj_ref, s_ref, b_ref, w2_ref, o_ref, adj16_ref):
    a16 = adj_ref[...].astype(jnp.bfloat16)
    adj16_ref[...] = a16
    z = jnp.dot(a16, s_ref[...], preferred_element_type=jnp.float32) + b_ref[...]
    z = _leaky(z)
    o_ref[...] = jnp.dot(z, w2_ref[...],
                         preferred_element_type=jnp.float32).astype(jnp.bfloat16)


def _agg_conv(adj, s, b, w2):
    return pl.pallas_call(
        _agg_conv_body,
        grid=(_N // _BM,),
        in_specs=[
            pl.BlockSpec((_BM, _N), lambda i: (i, 0)),
            pl.BlockSpec((_N, _D), lambda i: (0, 0)),
            pl.BlockSpec((1, _D), lambda i: (0, 0)),
            pl.BlockSpec((_D, _D), lambda i: (0, 0)),
        ],
        out_specs=[
            pl.BlockSpec((_BM, _D), lambda i: (i, 0)),
            pl.BlockSpec((_BM, _N), lambda i: (i, 0)),
        ],
        out_shape=[
            jax.ShapeDtypeStruct((_N, _D), jnp.bfloat16),
            jax.ShapeDtypeStruct((_N, _N), jnp.bfloat16),
        ],
    )(adj, s, b.reshape(1, _D), w2)


def _agg_relu_sup_body(adj_ref, s_ref, b_ref, w2_ref, act_ref, sup_ref):
    z = jnp.dot(adj_ref[...], s_ref[...],
                preferred_element_type=jnp.float32) + b_ref[...]
    z = jnp.maximum(z, 0.0)
    act_ref[...] = z
    sup_ref[...] = jnp.dot(z, w2_ref[...],
                           preferred_element_type=jnp.float32).astype(jnp.bfloat16)


def _agg_relu_sup(adj, s, b, w2):
    return pl.pallas_call(
        _agg_relu_sup_body,
        grid=(_N // _BM,),
        in_specs=[
            pl.BlockSpec((_BM, _N), lambda i: (i, 0)),
            pl.BlockSpec((_N, _D), lambda i: (0, 0)),
            pl.BlockSpec((1, _D), lambda i: (0, 0)),
            pl.BlockSpec((_D, _D), lambda i: (0, 0)),
        ],
        out_specs=[
            pl.BlockSpec((_BM, _D), lambda i: (i, 0)),
            pl.BlockSpec((_BM, _D), lambda i: (i, 0)),
        ],
        out_shape=[
            jax.ShapeDtypeStruct((_N, _D), jnp.float32),
            jax.ShapeDtypeStruct((_N, _D), jnp.bfloat16),
        ],
    )(adj, s, b.reshape(1, _D), w2)


def _agg_chain_body(adj_ref, s_ref, b_ref, w2_ref, o_ref):
    z = jnp.dot(adj_ref[...], s_ref[...],
                preferred_element_type=jnp.float32) + b_ref[...]
    z = _leaky(z)
    o_ref[...] = jnp.dot(z, w2_ref[...],
                         preferred_element_type=jnp.float32).astype(jnp.bfloat16)


def _agg_chain(adj, s, b, w2):
    return pl.pallas_call(
        _agg_chain_body,
        grid=(_N // _BM,),
        in_specs=[
            pl.BlockSpec((_BM, _N), lambda i: (i, 0)),
            pl.BlockSpec((_N, _D), lambda i: (0, 0)),
            pl.BlockSpec((1, _D), lambda i: (0, 0)),
            pl.BlockSpec((_D, _D), lambda i: (0, 0)),
        ],
        out_specs=pl.BlockSpec((_BM, _D), lambda i: (i, 0)),
        out_shape=jax.ShapeDtypeStruct((_N, _D), jnp.bfloat16),
    )(adj, s, b.reshape(1, _D), w2)


def _agg_mean_body(adj_ref, s_ref, b_ref, x0_ref, x1_ref, o_ref):
    z = jnp.dot(adj_ref[...], s_ref[...],
                preferred_element_type=jnp.float32) + b_ref[...]
    z = jnp.maximum(z, 0.0)
    o_ref[...] = (x0_ref[...] + x1_ref[...] + z) * (1.0 / 3.0)


def _agg_mean(adj, s, b, x0, x1):
    return pl.pallas_call(
        _agg_mean_body,
        grid=(_N // _BM,),
        in_specs=[
            pl.BlockSpec((_BM, _N), lambda i: (i, 0)),
            pl.BlockSpec((_N, _D), lambda i: (0, 0)),
            pl.BlockSpec((1, _D), lambda i: (0, 0)),
            pl.BlockSpec((_BM, _D), lambda i: (i, 0)),
            pl.BlockSpec((_BM, _D), lambda i: (i, 0)),
        ],
        out_specs=pl.BlockSpec((_BM, _D), lambda i: (i, 0)),
        out_shape=jax.ShapeDtypeStruct((_N, _D), jnp.float32),
    )(adj, s, b.reshape(1, _D), x0, x1)


def kernel(ufea, vfea, UV_adj, VU_adj, params):
    p = params
    s_u = _support(ufea, p["W_0_0"])
    s_v = _support(vfea, p["W_0_1"])
    s_u, VU16 = _agg_conv(VU_adj, s_u, p["b_0_0"], p["W_0_2"])
    s_v, UV16 = _agg_conv(UV_adj, s_v, p["b_0_1"], p["W_0_3"])
    u1, s_u = _agg_relu_sup(UV16, s_u, p["b_0_2"], p["W_1_0"])
    v1, s_v = _agg_relu_sup(VU16, s_v, p["b_0_3"], p["W_1_1"])
    s_u = _agg_chain(VU16, s_u, p["b_1_0"], p["W_1_2"])
    s_v = _agg_chain(UV16, s_v, p["b_1_1"], p["W_1_3"])
    out_u = _agg_mean(UV16, s_u, p["b_1_2"], ufea, u1)
    out_v = _agg_mean(VU16, s_v, p["b_1_3"], vfea, v1)
    return (out_u, out_v)


# speedup vs baseline: 1.6261x; 1.3048x over previous
import jax
import jax.numpy as jnp
from jax.experimental import pallas as pl

_N = 10000
_D = 128
_ALPHA = 0.2
_BM = 400


def _support_body(x_ref, w_ref, o_ref):
    o_ref[...] = jnp.dot(x_ref[...], w_ref[...],
                         preferred_element_type=jnp.float32).astype(jnp.bfloat16)


def _support(x, w):
    return pl.pallas_call(
        _support_body,
        out_shape=jax.ShapeDtypeStruct((_N, _D), jnp.bfloat16),
    )(x, w)


def _leaky(z):
    return jnp.where(z >= 0, z, _ALPHA * z)


def _agg_conv_body(adj_ref, s_ref, b_ref, w2_ref, o_ref, adj16_ref):
    a16 = adj_ref[...].astype(jnp.bfloat16)
    adj16_ref[...] = a16
    z = jnp.dot(a16, s_ref[...], preferred_element_type=jnp.float32) + b_ref[...]
    o_ref[...] = jnp.dot(_leaky(z), w2_ref[...],
                         preferred_element_type=jnp.float32).astype(jnp.bfloat16)


def _agg_conv(adj, s, b, w2):
    return pl.pallas_call(
        _agg_conv_body,
        grid=(_N // _BM,),
        in_specs=[
            pl.BlockSpec((_BM, _N), lambda i: (i, 0)),
            pl.BlockSpec((_N, _D), lambda i: (0, 0)),
            pl.BlockSpec((1, _D), lambda i: (0, 0)),
            pl.BlockSpec((_D, _D), lambda i: (0, 0)),
        ],
        out_specs=[
            pl.BlockSpec((_BM, _D), lambda i: (i, 0)),
            pl.BlockSpec((_BM, _N), lambda i: (i, 0)),
        ],
        out_shape=[
            jax.ShapeDtypeStruct((_N, _D), jnp.bfloat16),
            jax.ShapeDtypeStruct((_N, _N), jnp.bfloat16),
        ],
    )(adj, s, b.reshape(1, _D), w2)


def _pair_ab_body(adj_ref, s_ref, b_ref, wa_ref, wb_ref,
                  sa_out_ref, act_ref, sb_out_ref):
    a16 = adj_ref[...].astype(jnp.bfloat16)
    z = jnp.dot(a16, s_ref[...], preferred_element_type=jnp.float32) + b_ref[...]
    za = _leaky(z[:, :_D])
    zb = jnp.maximum(z[:, _D:], 0.0)
    sa_out_ref[...] = jnp.dot(za, wa_ref[...],
                              preferred_element_type=jnp.float32).astype(jnp.bfloat16)
    act_ref[...] = zb
    sb_out_ref[...] = jnp.dot(zb, wb_ref[...],
                              preferred_element_type=jnp.float32).astype(jnp.bfloat16)


def _pair_ab(adj, sa, sb, ba, bb, wa, wb):
    s_cat = jnp.concatenate([sa, sb], axis=1)
    b_cat = jnp.concatenate([ba, bb]).reshape(1, 2 * _D)
    return pl.pallas_call(
        _pair_ab_body,
        grid=(_N // _BM,),
        in_specs=[
            pl.BlockSpec((_BM, _N), lambda i: (i, 0)),
            pl.BlockSpec((_N, 2 * _D), lambda i: (0, 0)),
            pl.BlockSpec((1, 2 * _D), lambda i: (0, 0)),
            pl.BlockSpec((_D, _D), lambda i: (0, 0)),
            pl.BlockSpec((_D, _D), lambda i: (0, 0)),
        ],
        out_specs=[
            pl.BlockSpec((_BM, _D), lambda i: (i, 0)),
            pl.BlockSpec((_BM, _D), lambda i: (i, 0)),
            pl.BlockSpec((_BM, _D), lambda i: (i, 0)),
        ],
        out_shape=[
            jax.ShapeDtypeStruct((_N, _D), jnp.bfloat16),
            jax.ShapeDtypeStruct((_N, _D), jnp.float32),
            jax.ShapeDtypeStruct((_N, _D), jnp.bfloat16),
        ],
    )(adj, s_cat, b_cat, wa, wb)


def _pair_amean_body(adj_ref, s_ref, b_ref, wa_ref, x0_ref, x1_ref,
                     sa_out_ref, mean_ref):
    a16 = adj_ref[...].astype(jnp.bfloat16)
    z = jnp.dot(a16, s_ref[...], preferred_element_type=jnp.float32) + b_ref[...]
    za = _leaky(z[:, :_D])
    zb = jnp.maximum(z[:, _D:], 0.0)
    sa_out_ref[...] = jnp.dot(za, wa_ref[...],
                              preferred_element_type=jnp.float32).astype(jnp.bfloat16)
    mean_ref[...] = (x0_ref[...] + x1_ref[...] + zb) * (1.0 / 3.0)


def _pair_amean(adj, sa, sb, ba, bb, wa, x0, x1):
    s_cat = jnp.concatenate([sa, sb], axis=1)
    b_cat = jnp.concatenate([ba, bb]).reshape(1, 2 * _D)
    return pl.pallas_call(
        _pair_amean_body,
        grid=(_N // _BM,),
        in_specs=[
            pl.BlockSpec((_BM, _N), lambda i: (i, 0)),
            pl.BlockSpec((_N, 2 * _D), lambda i: (0, 0)),
            pl.BlockSpec((1, 2 * _D), lambda i: (0, 0)),
            pl.BlockSpec((_D, _D), lambda i: (0, 0)),
            pl.BlockSpec((_BM, _D), lambda i: (i, 0)),
            pl.BlockSpec((_BM, _D), lambda i: (i, 0)),
        ],
        out_specs=[
            pl.BlockSpec((_BM, _D), lambda i: (i, 0)),
            pl.BlockSpec((_BM, _D), lambda i: (i, 0)),
        ],
        out_shape=[
            jax.ShapeDtypeStruct((_N, _D), jnp.bfloat16),
            jax.ShapeDtypeStruct((_N, _D), jnp.float32),
        ],
    )(adj, s_cat, b_cat, wa, x0, x1)


def _agg_mean_body(adj_ref, s_ref, b_ref, x0_ref, x1_ref, o_ref):
    z = jnp.dot(adj_ref[...], s_ref[...],
                preferred_element_type=jnp.float32) + b_ref[...]
    z = jnp.maximum(z, 0.0)
    o_ref[...] = (x0_ref[...] + x1_ref[...] + z) * (1.0 / 3.0)


def _agg_mean(adj, s, b, x0, x1):
    return pl.pallas_call(
        _agg_mean_body,
        grid=(_N // _BM,),
        in_specs=[
            pl.BlockSpec((_BM, _N), lambda i: (i, 0)),
            pl.BlockSpec((_N, _D), lambda i: (0, 0)),
            pl.BlockSpec((1, _D), lambda i: (0, 0)),
            pl.BlockSpec((_BM, _D), lambda i: (i, 0)),
            pl.BlockSpec((_BM, _D), lambda i: (i, 0)),
        ],
        out_specs=pl.BlockSpec((_BM, _D), lambda i: (i, 0)),
        out_shape=jax.ShapeDtypeStruct((_N, _D), jnp.float32),
    )(adj, s, b.reshape(1, _D), x0, x1)


def kernel(ufea, vfea, UV_adj, VU_adj, params):
    p = params
    s_u0 = _support(ufea, p["W_0_0"])
    s_v0 = _support(vfea, p["W_0_1"])
    s_u1, VU16 = _agg_conv(VU_adj, s_u0, p["b_0_0"], p["W_0_2"])
    s_v1, u1, s_u_l1 = _pair_ab(UV_adj, s_v0, s_u1,
                                p["b_0_1"], p["b_0_2"], p["W_0_3"], p["W_1_0"])
    s_u1p, v1, s_v_l1 = _pair_ab(VU16, s_u_l1, s_v1,
                                 p["b_1_0"], p["b_0_3"], p["W_1_2"], p["W_1_1"])
    s_v1p, out_u = _pair_amean(UV_adj, s_v_l1, s_u1p,
                               p["b_1_1"], p["b_1_2"], p["W_1_3"], ufea, u1)
    out_v = _agg_mean(VU16, s_v1p, p["b_1_3"], vfea, v1)
    return (out_u, out_v)
